# baseline (speedup 1.0000x reference)
import functools

import jax
import jax.numpy as jnp
from jax import lax
from jax.experimental import pallas as pl
from jax.experimental.pallas import tpu as pltpu
from jax.experimental.pallas import tpu_sc as plsc

_NC = 2
_NS = 16
_NW = _NC * _NS
_L = 16
_GB = 128


@functools.cache
def _make_sc_kernel(rows: int, d: int, npad: int, n: int):
    stripe_rows = rows // _NW
    mesh = plsc.VectorSubcoreMesh(core_axis_name="c", subcore_axis_name="s")
    cbuf = stripe_rows + _L

    @functools.partial(
        pl.kernel,
        mesh=mesh,
        compiler_params=pltpu.CompilerParams(needs_layout_passes=False),
        out_type=jax.ShapeDtypeStruct((rows, d), jnp.float32),
        scratch_types=[
            pltpu.VMEM((3 * npad,), jnp.int32),
            pltpu.VMEM((stripe_rows, d), jnp.float32),
            pltpu.VMEM((cbuf,), jnp.int32),
            pltpu.VMEM((cbuf,), jnp.int32),
            pltpu.VMEM((cbuf,), jnp.int32),
            pltpu.VMEM((_GB, d), jnp.float32),
            pltpu.SemaphoreType.DMA,
            pltpu.SemaphoreType.DMA,
            pltpu.SemaphoreType.DMA,
        ],
    )
    def blend(x_hbm, ents_hbm, out_hbm,
              ents, stripe, lidx, tgtc, magc, gbuf, sem_s, sem_e, sem_g):
        wid = lax.axis_index("s") * _NC + lax.axis_index("c")
        base = wid * stripe_rows

        stripe_in = pltpu.async_copy(
            x_hbm.at[pl.ds(base, stripe_rows)], stripe, sem_s)
        pltpu.async_copy(ents_hbm, ents, sem_e).wait()

        def scan_body(v, cnt):
            off = v * _L
            rvec = ents[pl.ds(off, _L)]
            lref = rvec - base
            lane = lax.iota(jnp.int32, _L)
            msk = (lref >= 0) & (lref < stripe_rows) & (off + lane < n)
            pos = cnt + plsc.cumsum(msk.astype(jnp.int32)) - 1
            plsc.store_scatter(lidx, [pos], lref, mask=msk)
            plsc.store_scatter(tgtc, [pos], ents[pl.ds(npad + off, _L)],
                               mask=msk)
            plsc.store_scatter(magc, [pos], ents[pl.ds(2 * npad + off, _L)],
                               mask=msk)
            pc = plsc.all_reduce_population_count(msk)
            return cnt + pc[0]

        cnt = lax.fori_loop(0, npad // _L, scan_body, 0)
        tgtc[pl.ds(cnt, _L)] = jnp.zeros((_L,), jnp.int32)

        stripe_in.wait()

        ngroups = _GB // _L

        def batch_body(b, _):
            b0 = b * _GB
            for g in range(ngroups):
                @pl.when(b0 + g * _L < cnt)
                def _():
                    iv = jnp.clip(tgtc[pl.ds(b0 + g * _L, _L)], 0, rows - 1)
                    pltpu.async_copy(
                        x_hbm.at[iv], gbuf.at[pl.ds(g * _L, _L)], sem_g)
            for g in range(ngroups):
                @pl.when(b0 + g * _L < cnt)
                def _():
                    pltpu.make_async_copy(
                        x_hbm.at[jnp.zeros((_L,), jnp.int32)],
                        gbuf.at[pl.ds(g * _L, _L)], sem_g).wait()

            def group_body(g, _):
                off = b0 + g * _L
                m16 = plsc.bitcast(magc[pl.ds(off, _L)], jnp.float32)
                lid16 = lidx[pl.ds(off, _L)]
                for k in range(_L):
                    @pl.when(off + k < cnt)
                    def _():
                        lid = lid16[k]
                        m = jnp.full((_L,), m16[k], jnp.float32)
                        gr = g * _L + k
                        for j in range(d // _L):
                            sl = pl.ds(j * _L, _L)
                            t = gbuf[gr, sl]
                            r = stripe[lid, sl]
                            stripe[lid, sl] = r + m * (t - r)
                return 0

            nactive = jnp.minimum(cnt - b0, _GB)
            lax.fori_loop(0, (nactive + _L - 1) // _L, group_body, 0)
            return 0

        lax.fori_loop(0, (cnt + _GB - 1) // _GB, batch_body, 0)

        pltpu.async_copy(stripe, out_hbm.at[pl.ds(base, stripe_rows)],
                         sem_s).wait()

    return blend


def kernel(x, y, ref_index, target_index, mag):
    del y
    rows, d = x.shape
    n = ref_index.shape[0]
    npad = (n + _L - 1) // _L * _L
    pad = npad - n
    mag_bits = lax.bitcast_convert_type(mag, jnp.int32)
    if pad:
        refs_p = jnp.concatenate([ref_index, ref_index[:pad]])
        tgts_p = jnp.concatenate([target_index, target_index[:pad]])
        mags_p = jnp.concatenate([mag_bits, mag_bits[:pad]])
    else:
        refs_p, tgts_p, mags_p = ref_index, target_index, mag_bits
    ents = jnp.concatenate([refs_p, tgts_p, mags_p])
    return _make_sc_kernel(rows, d, npad, n)(x, ents)


# speedup vs baseline: 1.0094x; 1.0094x over previous
import functools

import jax
import jax.numpy as jnp
from jax import lax
from jax.experimental import pallas as pl
from jax.experimental.pallas import tpu as pltpu
from jax.experimental.pallas import tpu_sc as plsc

_NC = 2
_NS = 16
_NW = _NC * _NS
_L = 16
_GB = 128


@functools.cache
def _make_sc_kernel(rows: int, d: int, npad: int, n: int):
    stripe_rows = rows // _NW
    mesh = plsc.VectorSubcoreMesh(core_axis_name="c", subcore_axis_name="s")
    cbuf = stripe_rows + _L

    @functools.partial(
        pl.kernel,
        mesh=mesh,
        compiler_params=pltpu.CompilerParams(needs_layout_passes=False),
        out_type=jax.ShapeDtypeStruct((rows, d), jnp.float32),
        scratch_types=[
            pltpu.VMEM((3 * npad,), jnp.int32),
            pltpu.VMEM((stripe_rows, d), jnp.float32),
            pltpu.VMEM((cbuf,), jnp.int32),
            pltpu.VMEM((cbuf,), jnp.int32),
            pltpu.VMEM((cbuf,), jnp.int32),
            pltpu.VMEM((_GB, d), jnp.float32),
            pltpu.SemaphoreType.DMA,
            pltpu.SemaphoreType.DMA,
            pltpu.SemaphoreType.DMA,
        ],
    )
    def blend(x_hbm, ents_hbm, out_hbm,
              ents, stripe, lidx, tgtc, magc, gbuf, sem_s, sem_e, sem_g):
        wid = lax.axis_index("s") * _NC + lax.axis_index("c")
        base = wid * stripe_rows

        stripe_in = pltpu.async_copy(
            x_hbm.at[pl.ds(base, stripe_rows)], stripe, sem_s)
        pltpu.async_copy(ents_hbm, ents, sem_e).wait()

        def scan_body(v, cnt):
            off = v * _L
            rvec = ents[pl.ds(off, _L)]
            lref = rvec - base
            lane = lax.iota(jnp.int32, _L)
            msk = (lref >= 0) & (lref < stripe_rows) & (off + lane < n)
            pos = cnt + plsc.cumsum(msk.astype(jnp.int32)) - 1
            plsc.store_scatter(lidx, [pos], lref, mask=msk)
            plsc.store_scatter(tgtc, [pos], ents[pl.ds(npad + off, _L)],
                               mask=msk)
            plsc.store_scatter(magc, [pos], ents[pl.ds(2 * npad + off, _L)],
                               mask=msk)
            pc = plsc.all_reduce_population_count(msk)
            return cnt + pc[0]

        with jax.named_scope("sel_scan"):
            cnt = lax.fori_loop(0, npad // _L, scan_body, 0)
        tgtc[pl.ds(cnt, _L)] = jnp.zeros((_L,), jnp.int32)

        ngroups = _GB // _L

        def fire_gathers(b0):
            for g in range(ngroups):
                @pl.when(b0 + g * _L < cnt)
                def _():
                    iv = jnp.clip(tgtc[pl.ds(b0 + g * _L, _L)], 0, rows - 1)
                    pltpu.async_copy(
                        x_hbm.at[iv], gbuf.at[pl.ds(g * _L, _L)], sem_g)

        fire_gathers(0)
        with jax.named_scope("stripe_in_wait"):
            stripe_in.wait()

        def batch_body(b, _):
            b0 = b * _GB
            pl.when(b > 0)(lambda: fire_gathers(b0))
            for g in range(ngroups):
                @pl.when(b0 + g * _L < cnt)
                def _():
                    pltpu.make_async_copy(
                        x_hbm.at[jnp.zeros((_L,), jnp.int32)],
                        gbuf.at[pl.ds(g * _L, _L)], sem_g).wait()

            def group_body(g, _):
                off = b0 + g * _L
                m16 = plsc.bitcast(magc[pl.ds(off, _L)], jnp.float32)
                lid16 = lidx[pl.ds(off, _L)]
                for k in range(_L):
                    @pl.when(off + k < cnt)
                    def _():
                        lid = lid16[k]
                        m = jnp.full((_L,), m16[k], jnp.float32)
                        gr = g * _L + k
                        for j in range(d // _L):
                            sl = pl.ds(j * _L, _L)
                            t = gbuf[gr, sl]
                            r = stripe[lid, sl]
                            stripe[lid, sl] = r + m * (t - r)
                return 0

            nactive = jnp.minimum(cnt - b0, _GB)
            lax.fori_loop(0, (nactive + _L - 1) // _L, group_body, 0)
            return 0

        with jax.named_scope("gather_blend"):
            lax.fori_loop(0, (cnt + _GB - 1) // _GB, batch_body, 0)

        with jax.named_scope("stripe_out"):
            pltpu.async_copy(stripe, out_hbm.at[pl.ds(base, stripe_rows)],
                             sem_s).wait()

    return blend


def kernel(x, y, ref_index, target_index, mag):
    del y
    rows, d = x.shape
    n = ref_index.shape[0]
    npad = (n + _L - 1) // _L * _L
    pad = npad - n
    mag_bits = lax.bitcast_convert_type(mag, jnp.int32)
    if pad:
        refs_p = jnp.concatenate([ref_index, ref_index[:pad]])
        tgts_p = jnp.concatenate([target_index, target_index[:pad]])
        mags_p = jnp.concatenate([mag_bits, mag_bits[:pad]])
    else:
        refs_p, tgts_p, mags_p = ref_index, target_index, mag_bits
    ents = jnp.concatenate([refs_p, tgts_p, mags_p])
    return _make_sc_kernel(rows, d, npad, n)(x, ents)


# speedup vs baseline: 1.4264x; 1.4131x over previous
import functools

import jax
import jax.numpy as jnp
from jax import lax
from jax.experimental import pallas as pl
from jax.experimental.pallas import tpu as pltpu
from jax.experimental.pallas import tpu_sc as plsc

_NC = 2
_NS = 16
_NW = _NC * _NS
_L = 16
_WAVE = 256


@functools.cache
def _make_sc_kernel(rows: int, d: int, n: int):
    stripe_rows = rows // _NW
    npad = (n + _L - 1) // _L * _L
    mesh = plsc.VectorSubcoreMesh(core_axis_name="c", subcore_axis_name="s")
    cbuf = stripe_rows + _L

    @functools.partial(
        pl.kernel,
        mesh=mesh,
        compiler_params=pltpu.CompilerParams(needs_layout_passes=False),
        out_type=jax.ShapeDtypeStruct((rows, d), jnp.float32),
        scratch_types=[
            pltpu.VMEM((npad,), jnp.int32),
            pltpu.VMEM((npad,), jnp.int32),
            pltpu.VMEM((npad,), jnp.float32),
            pltpu.VMEM((stripe_rows, d), jnp.float32),
            pltpu.VMEM((cbuf,), jnp.int32),
            pltpu.VMEM((cbuf,), jnp.int32),
            pltpu.VMEM((cbuf,), jnp.float32),
            pltpu.VMEM((_WAVE, d), jnp.float32),
            pltpu.SemaphoreType.DMA,
            pltpu.SemaphoreType.DMA,
            pltpu.SemaphoreType.DMA,
        ],
    )
    def blend(x_hbm, refs_hbm, tgts_hbm, mags_hbm, out_hbm,
              refs_v, tgts_v, mags_v, stripe, lidx, tgtc, magc, gbuf,
              sem_s, sem_e, sem_g):
        wid = lax.axis_index("s") * _NC + lax.axis_index("c")
        base = wid * stripe_rows

        stripe_in = pltpu.async_copy(
            x_hbm.at[pl.ds(base, stripe_rows)], stripe, sem_s)
        e1 = pltpu.async_copy(refs_hbm, refs_v.at[pl.ds(0, n)], sem_e)
        e2 = pltpu.async_copy(tgts_hbm, tgts_v.at[pl.ds(0, n)], sem_e)
        e3 = pltpu.async_copy(mags_hbm, mags_v.at[pl.ds(0, n)], sem_e)
        e1.wait()
        e2.wait()
        e3.wait()

        def scan_body(v, cnt):
            off = v * _L
            rvec = refs_v[pl.ds(off, _L)]
            lref = rvec - base
            lane = lax.iota(jnp.int32, _L)
            msk = (lref >= 0) & (lref < stripe_rows) & (off + lane < n)
            pos = cnt + plsc.cumsum(msk.astype(jnp.int32)) - 1
            plsc.store_scatter(lidx, [pos], lref, mask=msk)
            plsc.store_scatter(tgtc, [pos], tgts_v[pl.ds(off, _L)], mask=msk)
            plsc.store_scatter(magc, [pos], mags_v[pl.ds(off, _L)], mask=msk)
            pc = plsc.all_reduce_population_count(msk)
            return cnt + pc[0]

        with jax.named_scope("sel_scan"):
            cnt = lax.fori_loop(0, npad // _L, scan_body, 0)

        def wave_body(w, _):
            w0 = w * _WAVE
            wcnt = jnp.minimum(cnt - w0, _WAVE)
            ngrp = (wcnt + _L - 1) // _L

            def fire_body(g, _):
                off = w0 + g * _L
                tgt16 = tgtc[pl.ds(off, _L)]
                for k in range(_L):
                    @pl.when(off + k < cnt)
                    def _():
                        pltpu.async_copy(
                            x_hbm.at[pl.ds(tgt16[k], 1)],
                            gbuf.at[pl.ds(g * _L + k, 1)], sem_g)
                return 0

            with jax.named_scope("fire_rows"):
                lax.fori_loop(0, ngrp, fire_body, 0)

            @pl.when(w == 0)
            def _():
                with jax.named_scope("stripe_in_wait"):
                    stripe_in.wait()

            def drain_blend_body(g, _):
                off = w0 + g * _L
                for k in range(_L):
                    @pl.when(off + k < cnt)
                    def _():
                        pltpu.make_async_copy(
                            x_hbm.at[pl.ds(0, 1)],
                            gbuf.at[pl.ds(g * _L + k, 1)], sem_g).wait()
                m16 = magc[pl.ds(off, _L)]
                lid16 = lidx[pl.ds(off, _L)]
                for k in range(_L):
                    @pl.when(off + k < cnt)
                    def _():
                        lid = lid16[k]
                        m = jnp.full((_L,), m16[k], jnp.float32)
                        gr = g * _L + k
                        for j in range(d // _L):
                            sl = pl.ds(j * _L, _L)
                            t = gbuf[gr, sl]
                            r = stripe[lid, sl]
                            stripe[lid, sl] = r + m * (t - r)
                return 0

            with jax.named_scope("drain_blend"):
                lax.fori_loop(0, ngrp, drain_blend_body, 0)
            return 0

        nwaves = jnp.maximum((cnt + _WAVE - 1) // _WAVE, 1)
        lax.fori_loop(0, nwaves, wave_body, 0)

        with jax.named_scope("stripe_out"):
            pltpu.async_copy(stripe, out_hbm.at[pl.ds(base, stripe_rows)],
                             sem_s).wait()

    return blend


def kernel(x, y, ref_index, target_index, mag):
    del y
    rows, d = x.shape
    n = ref_index.shape[0]
    return _make_sc_kernel(rows, d, n)(x, ref_index, target_index, mag)


# speedup vs baseline: 1.4300x; 1.0025x over previous
import functools

import jax
import jax.numpy as jnp
from jax import lax
from jax.experimental import pallas as pl
from jax.experimental.pallas import tpu as pltpu
from jax.experimental.pallas import tpu_sc as plsc

_NC = 2
_NS = 16
_NW = _NC * _NS
_L = 16
_WAVE = 256


@functools.cache
def _make_sc_kernel(rows: int, d: int, n: int):
    stripe_rows = rows // _NW
    npad = (n + _L - 1) // _L * _L
    mesh = plsc.VectorSubcoreMesh(core_axis_name="c", subcore_axis_name="s")
    cbuf = stripe_rows + _L

    @functools.partial(
        pl.kernel,
        mesh=mesh,
        compiler_params=pltpu.CompilerParams(
            needs_layout_passes=False, skip_device_barrier=True),
        out_type=jax.ShapeDtypeStruct((rows, d), jnp.float32),
        scratch_types=[
            pltpu.VMEM((npad,), jnp.int32),
            pltpu.VMEM((npad,), jnp.int32),
            pltpu.VMEM((npad,), jnp.float32),
            pltpu.VMEM((stripe_rows, d), jnp.float32),
            pltpu.VMEM((cbuf,), jnp.int32),
            pltpu.VMEM((cbuf,), jnp.int32),
            pltpu.VMEM((cbuf,), jnp.float32),
            pltpu.VMEM((_WAVE, d), jnp.float32),
            pltpu.SemaphoreType.DMA,
            pltpu.SemaphoreType.DMA,
            pltpu.SemaphoreType.DMA,
        ],
    )
    def blend(x_hbm, refs_hbm, tgts_hbm, mags_hbm, out_hbm,
              refs_v, tgts_v, mags_v, stripe, lidx, tgtc, magc, gbuf,
              sem_s, sem_e, sem_g):
        wid = lax.axis_index("s") * _NC + lax.axis_index("c")
        base = wid * stripe_rows

        stripe_in = pltpu.async_copy(
            x_hbm.at[pl.ds(base, stripe_rows)], stripe, sem_s)
        e1 = pltpu.async_copy(refs_hbm, refs_v.at[pl.ds(0, n)], sem_e)
        e2 = pltpu.async_copy(tgts_hbm, tgts_v.at[pl.ds(0, n)], sem_e)
        e3 = pltpu.async_copy(mags_hbm, mags_v.at[pl.ds(0, n)], sem_e)
        e1.wait()
        e2.wait()
        e3.wait()

        def scan_body(v, cnt):
            off = v * _L
            rvec = refs_v[pl.ds(off, _L)]
            lref = rvec - base
            lane = lax.iota(jnp.int32, _L)
            msk = (lref >= 0) & (lref < stripe_rows) & (off + lane < n)
            pos = cnt + plsc.cumsum(msk.astype(jnp.int32)) - 1
            plsc.store_scatter(lidx, [pos], lref, mask=msk)
            plsc.store_scatter(tgtc, [pos], tgts_v[pl.ds(off, _L)], mask=msk)
            plsc.store_scatter(magc, [pos], mags_v[pl.ds(off, _L)], mask=msk)
            pc = plsc.all_reduce_population_count(msk)
            return cnt + pc[0]

        with jax.named_scope("sel_scan"):
            cnt = lax.fori_loop(0, npad // _L, scan_body, 0)

        def wave_body(w, _):
            w0 = w * _WAVE
            wcnt = jnp.minimum(cnt - w0, _WAVE)
            ngrp = (wcnt + _L - 1) // _L

            def fire_body(g, _):
                off = w0 + g * _L
                tgt16 = tgtc[pl.ds(off, _L)]
                for k in range(_L):
                    @pl.when(off + k < cnt)
                    def _():
                        pltpu.async_copy(
                            x_hbm.at[pl.ds(tgt16[k], 1)],
                            gbuf.at[pl.ds(g * _L + k, 1)], sem_g)
                return 0

            with jax.named_scope("fire_rows"):
                lax.fori_loop(0, ngrp, fire_body, 0)

            @pl.when(w == 0)
            def _():
                with jax.named_scope("stripe_in_wait"):
                    stripe_in.wait()

            def drain_blend_body(g, _):
                off = w0 + g * _L
                for k in range(_L):
                    @pl.when(off + k < cnt)
                    def _():
                        pltpu.make_async_copy(
                            x_hbm.at[pl.ds(0, 1)],
                            gbuf.at[pl.ds(g * _L + k, 1)], sem_g).wait()
                m16 = magc[pl.ds(off, _L)]
                lid16 = lidx[pl.ds(off, _L)]
                for k in range(_L):
                    @pl.when(off + k < cnt)
                    def _():
                        lid = lid16[k]
                        m = jnp.full((_L,), m16[k], jnp.float32)
                        gr = g * _L + k
                        for j in range(d // _L):
                            sl = pl.ds(j * _L, _L)
                            t = gbuf[gr, sl]
                            r = stripe[lid, sl]
                            stripe[lid, sl] = r + m * (t - r)
                return 0

            with jax.named_scope("drain_blend"):
                lax.fori_loop(0, ngrp, drain_blend_body, 0)
            return 0

        nwaves = jnp.maximum((cnt + _WAVE - 1) // _WAVE, 1)
        lax.fori_loop(0, nwaves, wave_body, 0)

        with jax.named_scope("stripe_out"):
            pltpu.async_copy(stripe, out_hbm.at[pl.ds(base, stripe_rows)],
                             sem_s).wait()

    return blend


def kernel(x, y, ref_index, target_index, mag):
    del y
    rows, d = x.shape
    n = ref_index.shape[0]
    return _make_sc_kernel(rows, d, n)(x, ref_index, target_index, mag)
